# Initial kernel scaffold; baseline (speedup 1.0000x reference)
import jax
import jax.numpy as jnp
from jax.experimental import pallas as pl


def kernel(v0, v1):
    raise NotImplementedError("write your pallas kernel here")


# speedup vs baseline: 1.5862x; 1.5862x over previous
import functools

import jax
import jax.numpy as jnp
from jax import lax
from jax.experimental import pallas as pl
from jax.experimental.pallas import tpu as pltpu
from jax.experimental.pallas import tpu_sc as plsc

_B = 16384
_D = 128
_F = 13
_W = _F * _D


def _build_plan():
    plan = []
    off = [0, 0]
    for f in range(2 * _F):
        in_idx, c = divmod(f, _F)
        o = f % 2
        plan.append((o, off[o], in_idx, c))
        off[o] += 1
    return tuple(plan)


_PLAN = _build_plan()

_NC = 2
_NS = 16
_NW = _NC * _NS
_RPW = _B // _NW
_R = 16
_CHUNKS = _RPW // _R


def _permute_body(v0_hbm, v1_hbm, o0_hbm, o1_hbm, b0, b1, sem_in, sem_out):
    wid = lax.axis_index("s") * _NC + lax.axis_index("c")
    base = wid * _RPW
    srcs = (v0_hbm, v1_hbm)
    bufs = (b0, b1)
    outs = (o0_hbm, o1_hbm)

    def chunk(i, carry):
        r0 = base + i * _R
        handles = []
        for (o, j, s, c) in _PLAN:
            handles.append(pltpu.async_copy(
                srcs[s].at[pl.ds(r0, _R), pl.ds(c * _D, _D)],
                bufs[o].at[:, pl.ds(j * _D, _D)],
                sem_in))
        for h in handles:
            h.wait()
        wr = [pltpu.async_copy(bufs[o], outs[o].at[pl.ds(r0, _R), :], sem_out)
              for o in range(2)]
        for h in wr:
            h.wait()
        return carry

    lax.fori_loop(0, _CHUNKS, chunk, 0)


@functools.partial(
    pl.kernel,
    mesh=plsc.VectorSubcoreMesh(core_axis_name="c", subcore_axis_name="s"),
    out_type=(
        jax.ShapeDtypeStruct((_B, _W), jnp.float32),
        jax.ShapeDtypeStruct((_B, _W), jnp.float32),
    ),
    scratch_types=[
        pltpu.VMEM((_R, _W), jnp.float32),
        pltpu.VMEM((_R, _W), jnp.float32),
        pltpu.SemaphoreType.DMA,
        pltpu.SemaphoreType.DMA,
    ],
)
def _permute_sc(v0_hbm, v1_hbm, o0_hbm, o1_hbm, b0, b1, sem_in, sem_out):
    _permute_body(v0_hbm, v1_hbm, o0_hbm, o1_hbm, b0, b1, sem_in, sem_out)


def kernel(v0, v1):
    return _permute_sc(v0, v1)
